# Initial kernel scaffold; baseline (speedup 1.0000x reference)
import jax
import jax.numpy as jnp
from jax.experimental import pallas as pl


def kernel(cll, x_mol, edge_index, batch, W_rel1, b_rel1, W_root1, W_rel2, b_rel2, W_root2, W_mol, b_mol, W1, b1, W2, b2, W3, b3, W4, b4, Wc1, bc1, Wc2, bc2, Wc3, bc3):
    raise NotImplementedError("write your pallas kernel here")


# speedup vs baseline: 1.4095x; 1.4095x over previous
import functools

import jax
import jax.numpy as jnp
from jax import lax
from jax.experimental import pallas as pl
from jax.experimental.pallas import tpu as pltpu
from jax.experimental.pallas import tpu_sc as plsc

_N_NODES = 10000
_N_EDGES = 320000
_N_GRAPHS = 1024


def _dot(a, b):
    return jnp.dot(a, b, preferred_element_type=jnp.float32)


def _cll_body(x, w1, b1, w2, b2, w3, b3, w4, b4, out):
    c = jnp.maximum(_dot(x[...], w1[...]) + b1[...], 0.0)
    c = jnp.maximum(_dot(c, w2[...]) + b2[...], 0.0)
    c = jnp.maximum(_dot(c, w3[...]) + b3[...], 0.0)
    out[...] = _dot(c, w4[...]) + b4[...]


def _cll_mlp(cll, W1, b1, W2, b2, W3, b3, W4, b4):
    M, MB = _N_GRAPHS, 128
    full = lambda shape: pl.BlockSpec(shape, lambda i: (0, 0))
    return pl.pallas_call(
        _cll_body,
        grid=(M // MB,),
        in_specs=[
            pl.BlockSpec((MB, 4096), lambda i: (i, 0)),
            full((4096, 2000)), full((1, 2000)),
            full((2000, 1000)), full((1, 1000)),
            full((1000, 500)), full((1, 500)),
            full((500, 200)), full((1, 200)),
        ],
        out_specs=pl.BlockSpec((MB, 200), lambda i: (i, 0)),
        out_shape=jax.ShapeDtypeStruct((M, 200), jnp.float32),
    )(cll, W1, b1.reshape(1, -1), W2, b2.reshape(1, -1),
      W3, b3.reshape(1, -1), W4, b4.reshape(1, -1))


def _conv1_body(aggp, x, wcat, b, wrel2, h_out, hw2_out):
    agg = aggp[0] + aggp[1]
    xin = jnp.concatenate([agg, x[...]], axis=1)
    h = jnp.maximum(_dot(xin, wcat[...]) + b[...], 0.0)
    h_out[...] = h
    hw2_out[...] = _dot(h, wrel2[...])


def _conv1(agg1p, x_mol, Wcat1, b_rel1, W_rel2):
    M, MB = _N_NODES, 2000
    full = lambda shape: pl.BlockSpec(shape, lambda i: (0, 0))
    return pl.pallas_call(
        _conv1_body,
        grid=(M // MB,),
        in_specs=[
            pl.BlockSpec((2, MB, 128), lambda i: (0, i, 0)),
            pl.BlockSpec((MB, 128), lambda i: (i, 0)),
            full((256, 500)), full((1, 500)), full((500, 200)),
        ],
        out_specs=[
            pl.BlockSpec((MB, 500), lambda i: (i, 0)),
            pl.BlockSpec((MB, 200), lambda i: (i, 0)),
        ],
        out_shape=[
            jax.ShapeDtypeStruct((M, 500), jnp.float32),
            jax.ShapeDtypeStruct((M, 200), jnp.float32),
        ],
    )(agg1p, x_mol, Wcat1, b_rel1.reshape(1, -1), W_rel2)


def _conv2_body(agg2p, h, wroot2, b, h2_out):
    agg = agg2p[0] + agg2p[1]
    h2_out[...] = jnp.maximum(agg + b[...] + _dot(h[...], wroot2[...]), 0.0)


def _conv2(agg2p, h, W_root2, b_rel2):
    M, MB = _N_NODES, 2000
    full = lambda shape: pl.BlockSpec(shape, lambda i: (0, 0))
    return pl.pallas_call(
        _conv2_body,
        grid=(M // MB,),
        in_specs=[
            pl.BlockSpec((2, MB, 200), lambda i: (0, i, 0)),
            pl.BlockSpec((MB, 500), lambda i: (i, 0)),
            full((500, 200)), full((1, 200)),
        ],
        out_specs=pl.BlockSpec((MB, 200), lambda i: (i, 0)),
        out_shape=jax.ShapeDtypeStruct((M, 200), jnp.float32),
    )(agg2p, h, W_root2, b_rel2.reshape(1, -1))


def _final_body(poolp, cntp, c, wmol, bmol, wc1, bc1, wc2, bc2, wc3, bc3, out):
    sums = poolp[0] + poolp[1]
    cnt = cntp[0, :, 0:1] + cntp[1, :, 0:1]
    pooled = sums / jnp.maximum(cnt, 1.0)
    m = _dot(pooled, wmol[...]) + bmol[...]
    xcat = jnp.concatenate([m, c[...]], axis=1)
    z = jnp.maximum(_dot(xcat, wc1[...]) + bc1[...], 0.0)
    z = jnp.maximum(_dot(z, wc2[...]) + bc2[...], 0.0)
    out[...] = _dot(z, wc3[...]) + bc3[...]


def _final(poolp, cntp, c, W_mol, b_mol, Wc1, bc1, Wc2, bc2, Wc3, bc3):
    return pl.pallas_call(
        _final_body,
        out_shape=jax.ShapeDtypeStruct((_N_GRAPHS, 1), jnp.float32),
    )(poolp, cntp, c, W_mol, b_mol.reshape(1, -1), Wc1, bc1.reshape(1, -1),
      Wc2, bc2.reshape(1, -1), Wc3, bc3.reshape(1, -1))


def kernel(cll, x_mol, edge_index, batch,
           W_rel1, b_rel1, W_root1,
           W_rel2, b_rel2, W_root2,
           W_mol, b_mol,
           W1, b1, W2, b2, W3, b3, W4, b4,
           Wc1, bc1, Wc2, bc2, Wc3, bc3):
    src = edge_index[0]
    dst = edge_index[1]

    c = _cll_mlp(cll, W1, b1, W2, b2, W3, b3, W4, b4)

    agg1 = jax.ops.segment_sum(x_mol[src], dst, num_segments=_N_NODES)
    agg1p = jnp.stack([agg1, jnp.zeros_like(agg1)])
    Wcat1 = jnp.concatenate([W_rel1, W_root1], axis=0)
    h, hw2 = _conv1(agg1p, x_mol, Wcat1, b_rel1, W_rel2)

    agg2 = jax.ops.segment_sum(hw2[src], dst, num_segments=_N_NODES)
    agg2p = jnp.stack([agg2, jnp.zeros_like(agg2)])
    h2 = _conv2(agg2p, h, W_root2, b_rel2)

    sums = jax.ops.segment_sum(h2, batch, num_segments=_N_GRAPHS)
    cnt = jax.ops.segment_sum(jnp.ones((_N_NODES,), jnp.float32), batch,
                              num_segments=_N_GRAPHS)
    poolp = jnp.stack([sums, jnp.zeros_like(sums)])
    cntp = jnp.stack([jnp.broadcast_to(cnt[:, None], (_N_GRAPHS, 16)),
                      jnp.zeros((_N_GRAPHS, 16), jnp.float32)])

    return _final(poolp, cntp, c, W_mol, b_mol, Wc1, bc1, Wc2, bc2, Wc3, bc3)


# speedup vs baseline: 3.0499x; 2.1637x over previous
import functools

import jax
import jax.numpy as jnp
from jax import lax
from jax.experimental import pallas as pl
from jax.experimental.pallas import tpu as pltpu
from jax.experimental.pallas import tpu_sc as plsc

_N_NODES = 10000
_N_EDGES = 320000
_N_GRAPHS = 1024


_NC, _NS = 2, 16
_ECH = 80


@functools.lru_cache(maxsize=None)
def _make_edge_agg(n_tab):
    V, E, D = _N_NODES, _N_EDGES, 128
    e_per_w = E // (_NC * _NS)
    n_chunks = e_per_w // _ECH
    zrows = 624
    mesh = plsc.VectorSubcoreMesh(core_axis_name="c", subcore_axis_name="s")

    @functools.partial(
        pl.kernel,
        out_type=[jax.ShapeDtypeStruct((_NC, V, D), jnp.float32)
                  for _ in range(n_tab)],
        mesh=mesh,
        scratch_types=[
            pltpu.VMEM((_ECH,), jnp.int32),
            pltpu.VMEM((_ECH,), jnp.int32),
            pltpu.VMEM((_ECH, D), jnp.float32),
            pltpu.VMEM_SHARED((V, D), jnp.float32),
            pltpu.SemaphoreType.DMA,
        ],
    )
    def body(*refs):
        tabs = refs[:n_tab]
        src_h, dst_h, zeros_h = refs[n_tab:n_tab + 3]
        outs = refs[n_tab + 3:2 * n_tab + 3]
        src_v, dst_v, rows_v, acc_sh, sem = refs[2 * n_tab + 3:]

        cid = lax.axis_index("c")
        sid = lax.axis_index("s")
        wid = cid * _NS + sid
        base = sid * zrows

        for t in range(n_tab):
            pltpu.sync_copy(zeros_h.at[pl.ds(0, zrows)],
                            acc_sh.at[pl.ds(base, zrows)])

            @pl.when(sid == _NS - 1)
            def _():
                pltpu.sync_copy(zeros_h.at[pl.ds(0, 16)],
                                acc_sh.at[pl.ds(_NS * zrows, 16)])

            plsc.subcore_barrier()

            def ebody(i, _):
                off = wid * e_per_w + i * _ECH
                pltpu.sync_copy(src_h.at[pl.ds(off, _ECH)], src_v)
                pltpu.sync_copy(dst_h.at[pl.ds(off, _ECH)], dst_v)
                pltpu.async_copy(tabs[t].at[src_v], rows_v, sem).wait()
                pltpu.async_copy(rows_v, acc_sh.at[dst_v], sem,
                                 add=True).wait()
                return 0

            lax.fori_loop(0, n_chunks, ebody, 0)
            plsc.subcore_barrier()

            pltpu.sync_copy(acc_sh.at[pl.ds(base, zrows)],
                            outs[t].at[cid, pl.ds(base, zrows)])

            @pl.when(sid == _NS - 1)
            def _():
                pltpu.sync_copy(acc_sh.at[pl.ds(_NS * zrows, 16)],
                                outs[t].at[cid, pl.ds(_NS * zrows, 16)])

            plsc.subcore_barrier()

    return body


@functools.lru_cache(maxsize=None)
def _make_pool():
    V, G = _N_NODES, _N_GRAPHS
    n_total = V // _ECH
    g_per_s = G // _NS
    mesh = plsc.VectorSubcoreMesh(core_axis_name="c", subcore_axis_name="s")

    @functools.partial(
        pl.kernel,
        out_type=[
            jax.ShapeDtypeStruct((_NC, G, 128), jnp.float32),
            jax.ShapeDtypeStruct((_NC, G, 128), jnp.float32),
            jax.ShapeDtypeStruct((_NC, G, 128), jnp.float32),
        ],
        mesh=mesh,
        scratch_types=[
            pltpu.VMEM((_ECH,), jnp.int32),
            pltpu.VMEM((_ECH, 128), jnp.float32),
            pltpu.VMEM((_ECH, 128), jnp.float32),
            pltpu.VMEM((_ECH, 128), jnp.float32),
            pltpu.VMEM_SHARED((G, 128), jnp.float32),
            pltpu.VMEM_SHARED((G, 128), jnp.float32),
            pltpu.VMEM_SHARED((G, 128), jnp.float32),
            pltpu.SemaphoreType.DMA,
        ],
    )
    def body(h2a_h, h2b_h, batch_h, zeros_h, ones_h, suma_h, sumb_h, cnt_h,
             bidx_v, rowsa_v, rowsb_v, ones_v, acca_sh, accb_sh, cnt_sh, sem):
        cid = lax.axis_index("c")
        sid = lax.axis_index("s")
        wid = cid * _NS + sid

        base = sid * g_per_s
        pltpu.sync_copy(zeros_h.at[pl.ds(0, g_per_s)],
                        acca_sh.at[pl.ds(base, g_per_s)])
        pltpu.sync_copy(zeros_h.at[pl.ds(0, g_per_s)],
                        accb_sh.at[pl.ds(base, g_per_s)])
        pltpu.sync_copy(zeros_h.at[pl.ds(0, g_per_s)],
                        cnt_sh.at[pl.ds(base, g_per_s)])
        pltpu.sync_copy(ones_h, ones_v)
        plsc.subcore_barrier()

        n_mine = jnp.where(wid < (n_total - 3 * _NC * _NS), 4, 3)

        def nbody(i, _):
            off = (wid + i * _NC * _NS) * _ECH
            pltpu.sync_copy(batch_h.at[pl.ds(off, _ECH)], bidx_v)
            pltpu.sync_copy(h2a_h.at[pl.ds(off, _ECH)], rowsa_v)
            pltpu.sync_copy(h2b_h.at[pl.ds(off, _ECH)], rowsb_v)
            pltpu.async_copy(rowsa_v, acca_sh.at[bidx_v], sem, add=True).wait()
            pltpu.async_copy(rowsb_v, accb_sh.at[bidx_v], sem, add=True).wait()
            pltpu.async_copy(ones_v, cnt_sh.at[bidx_v], sem, add=True).wait()
            return 0

        lax.fori_loop(0, n_mine, nbody, 0)
        plsc.subcore_barrier()

        pltpu.sync_copy(acca_sh.at[pl.ds(base, g_per_s)],
                        suma_h.at[cid, pl.ds(base, g_per_s)])
        pltpu.sync_copy(accb_sh.at[pl.ds(base, g_per_s)],
                        sumb_h.at[cid, pl.ds(base, g_per_s)])
        pltpu.sync_copy(cnt_sh.at[pl.ds(base, g_per_s)],
                        cnt_h.at[cid, pl.ds(base, g_per_s)])

    return body


def _dot(a, b):
    return jnp.dot(a, b, preferred_element_type=jnp.float32)


def _cll_body(x, w1, b1, w2, b2, w3, b3, w4, b4, out):
    c = jnp.maximum(_dot(x[...], w1[...]) + b1[...], 0.0)
    c = jnp.maximum(_dot(c, w2[...]) + b2[...], 0.0)
    c = jnp.maximum(_dot(c, w3[...]) + b3[...], 0.0)
    out[...] = _dot(c, w4[...]) + b4[...]


def _cll_mlp(cll, W1, b1, W2, b2, W3, b3, W4, b4):
    M, MB = _N_GRAPHS, 128
    full = lambda shape: pl.BlockSpec(shape, lambda i: (0, 0))
    return pl.pallas_call(
        _cll_body,
        grid=(M // MB,),
        in_specs=[
            pl.BlockSpec((MB, 4096), lambda i: (i, 0)),
            full((4096, 2000)), full((1, 2000)),
            full((2000, 1000)), full((1, 1000)),
            full((1000, 500)), full((1, 500)),
            full((500, 200)), full((1, 200)),
        ],
        out_specs=pl.BlockSpec((MB, 200), lambda i: (i, 0)),
        out_shape=jax.ShapeDtypeStruct((M, 200), jnp.float32),
    )(cll, W1, b1.reshape(1, -1), W2, b2.reshape(1, -1),
      W3, b3.reshape(1, -1), W4, b4.reshape(1, -1))


def _conv1_body(aggp, x, wcat, b, h0_out, h1_out, h2_out, h3_out):
    agg = aggp[0] + aggp[1]
    xin = jnp.concatenate([agg, x[...]], axis=1)
    h = jnp.maximum(_dot(xin, wcat[...]) + b[...], 0.0)
    hp = jnp.pad(h, ((0, 0), (0, 12)))
    h0_out[...] = hp[:, 0:128]
    h1_out[...] = hp[:, 128:256]
    h2_out[...] = hp[:, 256:384]
    h3_out[...] = hp[:, 384:512]


def _conv1(agg1p, x_mol, Wcat1, b_rel1):
    M, MB = _N_NODES, 2000
    full = lambda shape: pl.BlockSpec(shape, lambda i: (0, 0))
    slab = lambda: pl.BlockSpec((MB, 128), lambda i: (i, 0))
    return pl.pallas_call(
        _conv1_body,
        grid=(M // MB,),
        in_specs=[
            pl.BlockSpec((2, MB, 128), lambda i: (0, i, 0)),
            pl.BlockSpec((MB, 128), lambda i: (i, 0)),
            full((256, 500)), full((1, 500)),
        ],
        out_specs=[slab(), slab(), slab(), slab()],
        out_shape=[jax.ShapeDtypeStruct((M, 128), jnp.float32)
                   for _ in range(4)],
    )(agg1p, x_mol, Wcat1, b_rel1.reshape(1, -1))


def _conv2_body(p0, p1, p2, p3, h0, h1, h2, h3, wrel2, wroot2, b,
                h2a_out, h2b_out):
    agg2 = jnp.concatenate(
        [p0[0] + p0[1], p1[0] + p1[1], p2[0] + p2[1], p3[0] + p3[1]],
        axis=1)[:, 0:500]
    hcat = jnp.concatenate([h0[...], h1[...], h2[...], h3[...]],
                           axis=1)[:, 0:500]
    out = jnp.maximum(
        _dot(agg2, wrel2[...]) + b[...] + _dot(hcat, wroot2[...]), 0.0)
    h2a_out[...] = out[:, 0:128]
    h2b_out[...] = out[:, 128:256]


def _conv2(aggps, hs, W_rel2, W_root2, b_rel2):
    M, MB = _N_NODES, 2000
    full = lambda shape: pl.BlockSpec(shape, lambda i: (0, 0))
    slab = lambda: pl.BlockSpec((MB, 128), lambda i: (i, 0))
    return pl.pallas_call(
        _conv2_body,
        grid=(M // MB,),
        in_specs=(
            [pl.BlockSpec((2, MB, 128), lambda i: (0, i, 0))
             for _ in range(4)]
            + [slab() for _ in range(4)]
            + [full((500, 256)), full((500, 256)), full((1, 256))]
        ),
        out_specs=[slab(), slab()],
        out_shape=[jax.ShapeDtypeStruct((M, 128), jnp.float32)
                   for _ in range(2)],
    )(*aggps, *hs, jnp.pad(W_rel2, ((0, 0), (0, 56))),
      jnp.pad(W_root2, ((0, 0), (0, 56))),
      jnp.pad(b_rel2, (0, 56)).reshape(1, -1))


def _final_body(poolpa, poolpb, cntp, c, wmol, bmol, wc1, bc1, wc2, bc2, wc3,
                bc3, out):
    sums = jnp.concatenate(
        [poolpa[0] + poolpa[1], poolpb[0, :, 0:72] + poolpb[1, :, 0:72]],
        axis=1)
    cnt = cntp[0, :, 0:1] + cntp[1, :, 0:1]
    pooled = sums / jnp.maximum(cnt, 1.0)
    m = _dot(pooled, wmol[...]) + bmol[...]
    xcat = jnp.concatenate([m, c[...]], axis=1)
    z = jnp.maximum(_dot(xcat, wc1[...]) + bc1[...], 0.0)
    z = jnp.maximum(_dot(z, wc2[...]) + bc2[...], 0.0)
    out[...] = _dot(z, wc3[...]) + bc3[...]


def _final(poolpa, poolpb, cntp, c, W_mol, b_mol, Wc1, bc1, Wc2, bc2, Wc3, bc3):
    return pl.pallas_call(
        _final_body,
        out_shape=jax.ShapeDtypeStruct((_N_GRAPHS, 1), jnp.float32),
    )(poolpa, poolpb, cntp, c, W_mol, b_mol.reshape(1, -1),
      Wc1, bc1.reshape(1, -1), Wc2, bc2.reshape(1, -1),
      Wc3, bc3.reshape(1, -1))


def kernel(cll, x_mol, edge_index, batch,
           W_rel1, b_rel1, W_root1,
           W_rel2, b_rel2, W_root2,
           W_mol, b_mol,
           W1, b1, W2, b2, W3, b3, W4, b4,
           Wc1, bc1, Wc2, bc2, Wc3, bc3):
    src = edge_index[0]
    dst = edge_index[1]

    c = _cll_mlp(cll, W1, b1, W2, b2, W3, b3, W4, b4)

    z128 = jnp.zeros((640, 128), jnp.float32)
    ones128 = jnp.ones((_ECH, 128), jnp.float32)

    (agg1p,) = _make_edge_agg(1)(x_mol, src, dst, z128)
    Wcat1 = jnp.concatenate([W_rel1, W_root1], axis=0)
    hs = _conv1(agg1p, x_mol, Wcat1, b_rel1)

    aggps = _make_edge_agg(4)(*hs, src, dst, z128)
    h2a, h2b = _conv2(aggps, hs, W_rel2, W_root2, b_rel2)

    poolpa, poolpb, cntp = _make_pool()(h2a, h2b, batch, z128[:64], ones128)

    return _final(poolpa, poolpb, cntp, c, W_mol, b_mol, Wc1, bc1, Wc2, bc2,
                  Wc3, bc3)


# speedup vs baseline: 4.8346x; 1.5852x over previous
import functools

import jax
import jax.numpy as jnp
from jax import lax
from jax.experimental import pallas as pl
from jax.experimental.pallas import tpu as pltpu
from jax.experimental.pallas import tpu_sc as plsc

_N_NODES = 10000
_N_EDGES = 320000
_N_GRAPHS = 1024


_NC, _NS = 2, 16
_ECH = 80


@functools.lru_cache(maxsize=None)
def _make_edge_agg(n_tab):
    V, E, D = _N_NODES, _N_EDGES, 128
    e_per_w = E // (_NC * _NS)
    n_chunks = e_per_w // _ECH
    zrows = 624
    mesh = plsc.VectorSubcoreMesh(core_axis_name="c", subcore_axis_name="s")

    n_pairs = (n_chunks - 1) // 2

    @functools.partial(
        pl.kernel,
        out_type=[jax.ShapeDtypeStruct((_NC, V, D), jnp.float32)
                  for _ in range(n_tab)],
        mesh=mesh,
        scratch_types=[
            pltpu.VMEM((_ECH,), jnp.int32),
            pltpu.VMEM((_ECH,), jnp.int32),
            pltpu.VMEM((_ECH,), jnp.int32),
            pltpu.VMEM((_ECH,), jnp.int32),
            pltpu.VMEM((_ECH, D), jnp.float32),
            pltpu.VMEM((_ECH, D), jnp.float32),
            pltpu.VMEM_SHARED((V, D), jnp.float32),
            pltpu.SemaphoreType.DMA,
            pltpu.SemaphoreType.DMA,
            pltpu.SemaphoreType.DMA,
            pltpu.SemaphoreType.DMA,
        ],
    )
    def body(*refs):
        tabs = refs[:n_tab]
        src_h, dst_h, zeros_h = refs[n_tab:n_tab + 3]
        outs = refs[n_tab + 3:2 * n_tab + 3]
        (src0, dst0, src1, dst1, rows0, rows1, acc_sh,
         sg0, sg1, ss0, ss1) = refs[2 * n_tab + 3:]

        cid = lax.axis_index("c")
        sid = lax.axis_index("s")
        wid = cid * _NS + sid
        base = sid * zrows
        ebase = wid * e_per_w

        for t in range(n_tab):
            tab = tabs[t]
            pltpu.sync_copy(zeros_h.at[pl.ds(0, zrows)],
                            acc_sh.at[pl.ds(base, zrows)])

            @pl.when(sid == _NS - 1)
            def _():
                pltpu.sync_copy(zeros_h.at[pl.ds(0, 16)],
                                acc_sh.at[pl.ds(_NS * zrows, 16)])

            plsc.subcore_barrier()

            pltpu.sync_copy(src_h.at[pl.ds(ebase, _ECH)], src0)
            pltpu.sync_copy(dst_h.at[pl.ds(ebase, _ECH)], dst0)
            pltpu.async_copy(tab.at[src0], rows0, sg0)

            def jbody(j, _):
                off1 = ebase + (2 * j + 1) * _ECH
                off2 = ebase + (2 * j + 2) * _ECH

                @pl.when(j > 0)
                def _():
                    pltpu.make_async_copy(rows1, acc_sh.at[dst1], ss1).wait()

                pltpu.sync_copy(src_h.at[pl.ds(off1, _ECH)], src1)
                pltpu.sync_copy(dst_h.at[pl.ds(off1, _ECH)], dst1)
                pltpu.async_copy(tab.at[src1], rows1, sg1)

                pltpu.make_async_copy(tab.at[src0], rows0, sg0).wait()
                pltpu.async_copy(rows0, acc_sh.at[dst0], ss0, add=True)
                pltpu.make_async_copy(rows0, acc_sh.at[dst0], ss0).wait()

                pltpu.sync_copy(src_h.at[pl.ds(off2, _ECH)], src0)
                pltpu.sync_copy(dst_h.at[pl.ds(off2, _ECH)], dst0)
                pltpu.async_copy(tab.at[src0], rows0, sg0)

                pltpu.make_async_copy(tab.at[src1], rows1, sg1).wait()
                pltpu.async_copy(rows1, acc_sh.at[dst1], ss1, add=True)
                return 0

            lax.fori_loop(0, n_pairs, jbody, 0)

            pltpu.make_async_copy(rows1, acc_sh.at[dst1], ss1).wait()
            pltpu.make_async_copy(tab.at[src0], rows0, sg0).wait()
            pltpu.async_copy(rows0, acc_sh.at[dst0], ss0, add=True)
            pltpu.make_async_copy(rows0, acc_sh.at[dst0], ss0).wait()

            plsc.subcore_barrier()

            pltpu.sync_copy(acc_sh.at[pl.ds(base, zrows)],
                            outs[t].at[cid, pl.ds(base, zrows)])

            @pl.when(sid == _NS - 1)
            def _():
                pltpu.sync_copy(acc_sh.at[pl.ds(_NS * zrows, 16)],
                                outs[t].at[cid, pl.ds(_NS * zrows, 16)])

            plsc.subcore_barrier()

    return body


@functools.lru_cache(maxsize=None)
def _make_pool():
    V, G = _N_NODES, _N_GRAPHS
    n_total = V // _ECH
    g_per_s = G // _NS
    mesh = plsc.VectorSubcoreMesh(core_axis_name="c", subcore_axis_name="s")

    @functools.partial(
        pl.kernel,
        out_type=[
            jax.ShapeDtypeStruct((_NC, G, 128), jnp.float32),
            jax.ShapeDtypeStruct((_NC, G, 128), jnp.float32),
            jax.ShapeDtypeStruct((_NC, G, 128), jnp.float32),
        ],
        mesh=mesh,
        scratch_types=[
            pltpu.VMEM((_ECH,), jnp.int32),
            pltpu.VMEM((_ECH, 128), jnp.float32),
            pltpu.VMEM((_ECH, 128), jnp.float32),
            pltpu.VMEM((_ECH, 128), jnp.float32),
            pltpu.VMEM_SHARED((G, 128), jnp.float32),
            pltpu.VMEM_SHARED((G, 128), jnp.float32),
            pltpu.VMEM_SHARED((G, 128), jnp.float32),
            pltpu.SemaphoreType.DMA,
        ],
    )
    def body(h2a_h, h2b_h, batch_h, zeros_h, ones_h, suma_h, sumb_h, cnt_h,
             bidx_v, rowsa_v, rowsb_v, ones_v, acca_sh, accb_sh, cnt_sh, sem):
        cid = lax.axis_index("c")
        sid = lax.axis_index("s")
        wid = cid * _NS + sid

        base = sid * g_per_s
        pltpu.sync_copy(zeros_h.at[pl.ds(0, g_per_s)],
                        acca_sh.at[pl.ds(base, g_per_s)])
        pltpu.sync_copy(zeros_h.at[pl.ds(0, g_per_s)],
                        accb_sh.at[pl.ds(base, g_per_s)])
        pltpu.sync_copy(zeros_h.at[pl.ds(0, g_per_s)],
                        cnt_sh.at[pl.ds(base, g_per_s)])
        pltpu.sync_copy(ones_h, ones_v)
        plsc.subcore_barrier()

        n_mine = jnp.where(wid < (n_total - 3 * _NC * _NS), 4, 3)

        def nbody(i, _):
            off = (wid + i * _NC * _NS) * _ECH
            pltpu.sync_copy(batch_h.at[pl.ds(off, _ECH)], bidx_v)
            pltpu.sync_copy(h2a_h.at[pl.ds(off, _ECH)], rowsa_v)
            pltpu.sync_copy(h2b_h.at[pl.ds(off, _ECH)], rowsb_v)
            pltpu.async_copy(rowsa_v, acca_sh.at[bidx_v], sem, add=True).wait()
            pltpu.async_copy(rowsb_v, accb_sh.at[bidx_v], sem, add=True).wait()
            pltpu.async_copy(ones_v, cnt_sh.at[bidx_v], sem, add=True).wait()
            return 0

        lax.fori_loop(0, n_mine, nbody, 0)
        plsc.subcore_barrier()

        pltpu.sync_copy(acca_sh.at[pl.ds(base, g_per_s)],
                        suma_h.at[cid, pl.ds(base, g_per_s)])
        pltpu.sync_copy(accb_sh.at[pl.ds(base, g_per_s)],
                        sumb_h.at[cid, pl.ds(base, g_per_s)])
        pltpu.sync_copy(cnt_sh.at[pl.ds(base, g_per_s)],
                        cnt_h.at[cid, pl.ds(base, g_per_s)])

    return body


def _dot(a, b):
    return jnp.dot(a, b, preferred_element_type=jnp.float32)


def _cll_body(x, w1, b1, w2, b2, w3, b3, w4, b4, out):
    c = jnp.maximum(_dot(x[...], w1[...]) + b1[...], 0.0)
    c = jnp.maximum(_dot(c, w2[...]) + b2[...], 0.0)
    c = jnp.maximum(_dot(c, w3[...]) + b3[...], 0.0)
    out[...] = _dot(c, w4[...]) + b4[...]


def _cll_mlp(cll, W1, b1, W2, b2, W3, b3, W4, b4):
    M, MB = _N_GRAPHS, 128
    full = lambda shape: pl.BlockSpec(shape, lambda i: (0, 0))
    return pl.pallas_call(
        _cll_body,
        grid=(M // MB,),
        in_specs=[
            pl.BlockSpec((MB, 4096), lambda i: (i, 0)),
            full((4096, 2000)), full((1, 2000)),
            full((2000, 1000)), full((1, 1000)),
            full((1000, 500)), full((1, 500)),
            full((500, 200)), full((1, 200)),
        ],
        out_specs=pl.BlockSpec((MB, 200), lambda i: (i, 0)),
        out_shape=jax.ShapeDtypeStruct((M, 200), jnp.float32),
    )(cll, W1, b1.reshape(1, -1), W2, b2.reshape(1, -1),
      W3, b3.reshape(1, -1), W4, b4.reshape(1, -1))


def _conv1_body(aggp, x, wcat, b, h0_out, h1_out, h2_out, h3_out):
    agg = aggp[0] + aggp[1]
    xin = jnp.concatenate([agg, x[...]], axis=1)
    h = jnp.maximum(_dot(xin, wcat[...]) + b[...], 0.0)
    hp = jnp.pad(h, ((0, 0), (0, 12)))
    h0_out[...] = hp[:, 0:128]
    h1_out[...] = hp[:, 128:256]
    h2_out[...] = hp[:, 256:384]
    h3_out[...] = hp[:, 384:512]


def _conv1(agg1p, x_mol, Wcat1, b_rel1):
    M, MB = _N_NODES, 2000
    full = lambda shape: pl.BlockSpec(shape, lambda i: (0, 0))
    slab = lambda: pl.BlockSpec((MB, 128), lambda i: (i, 0))
    return pl.pallas_call(
        _conv1_body,
        grid=(M // MB,),
        in_specs=[
            pl.BlockSpec((2, MB, 128), lambda i: (0, i, 0)),
            pl.BlockSpec((MB, 128), lambda i: (i, 0)),
            full((256, 500)), full((1, 500)),
        ],
        out_specs=[slab(), slab(), slab(), slab()],
        out_shape=[jax.ShapeDtypeStruct((M, 128), jnp.float32)
                   for _ in range(4)],
    )(agg1p, x_mol, Wcat1, b_rel1.reshape(1, -1))


def _conv2_body(p0, p1, p2, p3, h0, h1, h2, h3, wrel2, wroot2, b,
                h2a_out, h2b_out):
    agg2 = jnp.concatenate(
        [p0[0] + p0[1], p1[0] + p1[1], p2[0] + p2[1], p3[0] + p3[1]],
        axis=1)[:, 0:500]
    hcat = jnp.concatenate([h0[...], h1[...], h2[...], h3[...]],
                           axis=1)[:, 0:500]
    out = jnp.maximum(
        _dot(agg2, wrel2[...]) + b[...] + _dot(hcat, wroot2[...]), 0.0)
    h2a_out[...] = out[:, 0:128]
    h2b_out[...] = out[:, 128:256]


def _conv2(aggps, hs, W_rel2, W_root2, b_rel2):
    M, MB = _N_NODES, 2000
    full = lambda shape: pl.BlockSpec(shape, lambda i: (0, 0))
    slab = lambda: pl.BlockSpec((MB, 128), lambda i: (i, 0))
    return pl.pallas_call(
        _conv2_body,
        grid=(M // MB,),
        in_specs=(
            [pl.BlockSpec((2, MB, 128), lambda i: (0, i, 0))
             for _ in range(4)]
            + [slab() for _ in range(4)]
            + [full((500, 256)), full((500, 256)), full((1, 256))]
        ),
        out_specs=[slab(), slab()],
        out_shape=[jax.ShapeDtypeStruct((M, 128), jnp.float32)
                   for _ in range(2)],
    )(*aggps, *hs, jnp.pad(W_rel2, ((0, 0), (0, 56))),
      jnp.pad(W_root2, ((0, 0), (0, 56))),
      jnp.pad(b_rel2, (0, 56)).reshape(1, -1))


def _final_body(poolpa, poolpb, cntp, c, wmol, bmol, wc1, bc1, wc2, bc2, wc3,
                bc3, out):
    sums = jnp.concatenate(
        [poolpa[0] + poolpa[1], poolpb[0, :, 0:72] + poolpb[1, :, 0:72]],
        axis=1)
    cnt = cntp[0, :, 0:1] + cntp[1, :, 0:1]
    pooled = sums / jnp.maximum(cnt, 1.0)
    m = _dot(pooled, wmol[...]) + bmol[...]
    xcat = jnp.concatenate([m, c[...]], axis=1)
    z = jnp.maximum(_dot(xcat, wc1[...]) + bc1[...], 0.0)
    z = jnp.maximum(_dot(z, wc2[...]) + bc2[...], 0.0)
    out[...] = _dot(z, wc3[...]) + bc3[...]


def _final(poolpa, poolpb, cntp, c, W_mol, b_mol, Wc1, bc1, Wc2, bc2, Wc3, bc3):
    return pl.pallas_call(
        _final_body,
        out_shape=jax.ShapeDtypeStruct((_N_GRAPHS, 1), jnp.float32),
    )(poolpa, poolpb, cntp, c, W_mol, b_mol.reshape(1, -1),
      Wc1, bc1.reshape(1, -1), Wc2, bc2.reshape(1, -1),
      Wc3, bc3.reshape(1, -1))


def kernel(cll, x_mol, edge_index, batch,
           W_rel1, b_rel1, W_root1,
           W_rel2, b_rel2, W_root2,
           W_mol, b_mol,
           W1, b1, W2, b2, W3, b3, W4, b4,
           Wc1, bc1, Wc2, bc2, Wc3, bc3):
    src = edge_index[0]
    dst = edge_index[1]

    c = _cll_mlp(cll, W1, b1, W2, b2, W3, b3, W4, b4)

    z128 = jnp.zeros((640, 128), jnp.float32)
    ones128 = jnp.ones((_ECH, 128), jnp.float32)

    (agg1p,) = _make_edge_agg(1)(x_mol, src, dst, z128)
    Wcat1 = jnp.concatenate([W_rel1, W_root1], axis=0)
    hs = _conv1(agg1p, x_mol, Wcat1, b_rel1)

    aggps = _make_edge_agg(4)(*hs, src, dst, z128)
    h2a, h2b = _conv2(aggps, hs, W_rel2, W_root2, b_rel2)

    poolpa, poolpb, cntp = _make_pool()(h2a, h2b, batch, z128[:64], ones128)

    return _final(poolpa, poolpb, cntp, c, W_mol, b_mol, Wc1, bc1, Wc2, bc2,
                  Wc3, bc3)


# speedup vs baseline: 6.9211x; 1.4316x over previous
import functools

import jax
import jax.numpy as jnp
from jax import lax
from jax.experimental import pallas as pl
from jax.experimental.pallas import tpu as pltpu
from jax.experimental.pallas import tpu_sc as plsc

_N_NODES = 10000
_N_EDGES = 320000
_N_GRAPHS = 1024


_NC, _NS = 2, 16
_ECH = 80


@functools.lru_cache(maxsize=None)
def _make_edge_agg(n_tab):
    V, E, D = _N_NODES, _N_EDGES, 128
    e_per_w = E // (_NC * _NS)
    n_chunks = e_per_w // _ECH
    zrows = 624
    mesh = plsc.VectorSubcoreMesh(core_axis_name="c", subcore_axis_name="s")

    n_pairs = (n_chunks - 1) // 2

    @functools.partial(
        pl.kernel,
        out_type=[jax.ShapeDtypeStruct((_NC, V, D), jnp.float32)
                  for _ in range(n_tab)],
        mesh=mesh,
        scratch_types=[
            pltpu.VMEM((e_per_w,), jnp.int32),
            pltpu.VMEM((_ECH,), jnp.int32),
            pltpu.VMEM((_ECH,), jnp.int32),
            pltpu.VMEM((_ECH, D), jnp.float32),
            pltpu.VMEM((_ECH, D), jnp.float32),
            pltpu.VMEM_SHARED((V, D), jnp.float32),
            pltpu.SemaphoreType.DMA,
            pltpu.SemaphoreType.DMA,
            pltpu.SemaphoreType.DMA,
            pltpu.SemaphoreType.DMA,
            pltpu.SemaphoreType.DMA,
            pltpu.SemaphoreType.DMA,
        ],
    )
    def body(*refs):
        tabs = refs[:n_tab]
        src_h, dst_h, zeros_h = refs[n_tab:n_tab + 3]
        outs = refs[n_tab + 3:2 * n_tab + 3]
        (src_c, dst0, dst1, rows0, rows1, acc_sh,
         sg0, sg1, ss0, ss1, sd0, sd1) = refs[2 * n_tab + 3:]

        cid = lax.axis_index("c")
        sid = lax.axis_index("s")
        wid = cid * _NS + sid
        base = sid * zrows
        ebase = wid * e_per_w

        pltpu.sync_copy(src_h.at[pl.ds(ebase, e_per_w)], src_c)

        for t in range(n_tab):
            tab = tabs[t]
            pltpu.sync_copy(zeros_h.at[pl.ds(0, zrows)],
                            acc_sh.at[pl.ds(base, zrows)])

            @pl.when(sid == _NS - 1)
            def _():
                pltpu.sync_copy(zeros_h.at[pl.ds(0, 16)],
                                acc_sh.at[pl.ds(_NS * zrows, 16)])

            plsc.subcore_barrier()

            pltpu.async_copy(dst_h.at[pl.ds(ebase, _ECH)], dst0, sd0)
            pltpu.async_copy(tab.at[src_c.at[pl.ds(0, _ECH)]], rows0, sg0)

            def jbody(j, _):
                off1 = ebase + (2 * j + 1) * _ECH
                off2 = ebase + (2 * j + 2) * _ECH
                s1 = pl.ds((2 * j + 1) * _ECH, _ECH)
                s2 = pl.ds((2 * j + 2) * _ECH, _ECH)

                @pl.when(j > 0)
                def _():
                    pltpu.make_async_copy(rows1, acc_sh.at[dst1], ss1).wait()

                pltpu.async_copy(dst_h.at[pl.ds(off1, _ECH)], dst1, sd1)
                pltpu.async_copy(tab.at[src_c.at[s1]], rows1, sg1)

                pltpu.make_async_copy(tab.at[src_c.at[pl.ds(0, _ECH)]], rows0, sg0).wait()
                pltpu.make_async_copy(dst_h.at[pl.ds(off1, _ECH)], dst0,
                                      sd0).wait()
                pltpu.async_copy(rows0, acc_sh.at[dst0], ss0, add=True)
                pltpu.make_async_copy(rows0, acc_sh.at[dst0], ss0).wait()

                pltpu.async_copy(dst_h.at[pl.ds(off2, _ECH)], dst0, sd0)
                pltpu.async_copy(tab.at[src_c.at[s2]], rows0, sg0)

                pltpu.make_async_copy(tab.at[src_c.at[pl.ds(0, _ECH)]], rows1, sg1).wait()
                pltpu.make_async_copy(dst_h.at[pl.ds(off1, _ECH)], dst1,
                                      sd1).wait()
                pltpu.async_copy(rows1, acc_sh.at[dst1], ss1, add=True)
                return 0

            lax.fori_loop(0, n_pairs, jbody, 0)

            pltpu.make_async_copy(rows1, acc_sh.at[dst1], ss1).wait()
            pltpu.make_async_copy(tab.at[src_c.at[pl.ds(0, _ECH)]], rows0, sg0).wait()
            pltpu.make_async_copy(dst_h.at[pl.ds(ebase, _ECH)], dst0,
                                  sd0).wait()
            pltpu.async_copy(rows0, acc_sh.at[dst0], ss0, add=True)
            pltpu.make_async_copy(rows0, acc_sh.at[dst0], ss0).wait()

            plsc.subcore_barrier()

            pltpu.sync_copy(acc_sh.at[pl.ds(base, zrows)],
                            outs[t].at[cid, pl.ds(base, zrows)])

            @pl.when(sid == _NS - 1)
            def _():
                pltpu.sync_copy(acc_sh.at[pl.ds(_NS * zrows, 16)],
                                outs[t].at[cid, pl.ds(_NS * zrows, 16)])

            plsc.subcore_barrier()

    return body


@functools.lru_cache(maxsize=None)
def _make_pool():
    V, G = _N_NODES, _N_GRAPHS
    n_total = V // _ECH
    g_per_s = G // _NS
    mesh = plsc.VectorSubcoreMesh(core_axis_name="c", subcore_axis_name="s")

    @functools.partial(
        pl.kernel,
        out_type=[
            jax.ShapeDtypeStruct((_NC, G, 128), jnp.float32),
            jax.ShapeDtypeStruct((_NC, G, 128), jnp.float32),
            jax.ShapeDtypeStruct((_NC, G, 128), jnp.float32),
        ],
        mesh=mesh,
        scratch_types=[
            pltpu.VMEM((_ECH,), jnp.int32),
            pltpu.VMEM((_ECH, 128), jnp.float32),
            pltpu.VMEM((_ECH, 128), jnp.float32),
            pltpu.VMEM((_ECH, 128), jnp.float32),
            pltpu.VMEM_SHARED((G, 128), jnp.float32),
            pltpu.VMEM_SHARED((G, 128), jnp.float32),
            pltpu.VMEM_SHARED((G, 128), jnp.float32),
            pltpu.SemaphoreType.DMA,
        ],
    )
    def body(h2a_h, h2b_h, batch_h, zeros_h, ones_h, suma_h, sumb_h, cnt_h,
             bidx_v, rowsa_v, rowsb_v, ones_v, acca_sh, accb_sh, cnt_sh, sem):
        cid = lax.axis_index("c")
        sid = lax.axis_index("s")
        wid = cid * _NS + sid

        base = sid * g_per_s
        pltpu.sync_copy(zeros_h.at[pl.ds(0, g_per_s)],
                        acca_sh.at[pl.ds(base, g_per_s)])
        pltpu.sync_copy(zeros_h.at[pl.ds(0, g_per_s)],
                        accb_sh.at[pl.ds(base, g_per_s)])
        pltpu.sync_copy(zeros_h.at[pl.ds(0, g_per_s)],
                        cnt_sh.at[pl.ds(base, g_per_s)])
        pltpu.sync_copy(ones_h, ones_v)
        plsc.subcore_barrier()

        n_mine = jnp.where(wid < (n_total - 3 * _NC * _NS), 4, 3)

        def nbody(i, _):
            off = (wid + i * _NC * _NS) * _ECH
            pltpu.sync_copy(batch_h.at[pl.ds(off, _ECH)], bidx_v)
            pltpu.sync_copy(h2a_h.at[pl.ds(off, _ECH)], rowsa_v)
            pltpu.sync_copy(h2b_h.at[pl.ds(off, _ECH)], rowsb_v)
            pltpu.async_copy(rowsa_v, acca_sh.at[bidx_v], sem, add=True).wait()
            pltpu.async_copy(rowsb_v, accb_sh.at[bidx_v], sem, add=True).wait()
            pltpu.async_copy(ones_v, cnt_sh.at[bidx_v], sem, add=True).wait()
            return 0

        lax.fori_loop(0, n_mine, nbody, 0)
        plsc.subcore_barrier()

        pltpu.sync_copy(acca_sh.at[pl.ds(base, g_per_s)],
                        suma_h.at[cid, pl.ds(base, g_per_s)])
        pltpu.sync_copy(accb_sh.at[pl.ds(base, g_per_s)],
                        sumb_h.at[cid, pl.ds(base, g_per_s)])
        pltpu.sync_copy(cnt_sh.at[pl.ds(base, g_per_s)],
                        cnt_h.at[cid, pl.ds(base, g_per_s)])

    return body


def _dot(a, b):
    return jnp.dot(a, b, preferred_element_type=jnp.float32)


def _cll_body(x, w1, b1, w2, b2, w3, b3, w4, b4, out):
    c = jnp.maximum(_dot(x[...], w1[...]) + b1[...], 0.0)
    c = jnp.maximum(_dot(c, w2[...]) + b2[...], 0.0)
    c = jnp.maximum(_dot(c, w3[...]) + b3[...], 0.0)
    out[...] = _dot(c, w4[...]) + b4[...]


def _cll_mlp(cll, W1, b1, W2, b2, W3, b3, W4, b4):
    M, MB = _N_GRAPHS, 128
    full = lambda shape: pl.BlockSpec(shape, lambda i: (0, 0))
    return pl.pallas_call(
        _cll_body,
        grid=(M // MB,),
        in_specs=[
            pl.BlockSpec((MB, 4096), lambda i: (i, 0)),
            full((4096, 2000)), full((1, 2000)),
            full((2000, 1000)), full((1, 1000)),
            full((1000, 500)), full((1, 500)),
            full((500, 200)), full((1, 200)),
        ],
        out_specs=pl.BlockSpec((MB, 200), lambda i: (i, 0)),
        out_shape=jax.ShapeDtypeStruct((M, 200), jnp.float32),
    )(cll, W1, b1.reshape(1, -1), W2, b2.reshape(1, -1),
      W3, b3.reshape(1, -1), W4, b4.reshape(1, -1))


def _conv1_body(aggp, x, wcat, b, h0_out, h1_out, h2_out, h3_out):
    agg = aggp[0] + aggp[1]
    xin = jnp.concatenate([agg, x[...]], axis=1)
    h = jnp.maximum(_dot(xin, wcat[...]) + b[...], 0.0)
    hp = jnp.pad(h, ((0, 0), (0, 12)))
    h0_out[...] = hp[:, 0:128]
    h1_out[...] = hp[:, 128:256]
    h2_out[...] = hp[:, 256:384]
    h3_out[...] = hp[:, 384:512]


def _conv1(agg1p, x_mol, Wcat1, b_rel1):
    M, MB = _N_NODES, 2000
    full = lambda shape: pl.BlockSpec(shape, lambda i: (0, 0))
    slab = lambda: pl.BlockSpec((MB, 128), lambda i: (i, 0))
    return pl.pallas_call(
        _conv1_body,
        grid=(M // MB,),
        in_specs=[
            pl.BlockSpec((2, MB, 128), lambda i: (0, i, 0)),
            pl.BlockSpec((MB, 128), lambda i: (i, 0)),
            full((256, 500)), full((1, 500)),
        ],
        out_specs=[slab(), slab(), slab(), slab()],
        out_shape=[jax.ShapeDtypeStruct((M, 128), jnp.float32)
                   for _ in range(4)],
    )(agg1p, x_mol, Wcat1, b_rel1.reshape(1, -1))


def _conv2_body(p0, p1, p2, p3, h0, h1, h2, h3, wrel2, wroot2, b,
                h2a_out, h2b_out):
    agg2 = jnp.concatenate(
        [p0[0] + p0[1], p1[0] + p1[1], p2[0] + p2[1], p3[0] + p3[1]],
        axis=1)[:, 0:500]
    hcat = jnp.concatenate([h0[...], h1[...], h2[...], h3[...]],
                           axis=1)[:, 0:500]
    out = jnp.maximum(
        _dot(agg2, wrel2[...]) + b[...] + _dot(hcat, wroot2[...]), 0.0)
    h2a_out[...] = out[:, 0:128]
    h2b_out[...] = out[:, 128:256]


def _conv2(aggps, hs, W_rel2, W_root2, b_rel2):
    M, MB = _N_NODES, 2000
    full = lambda shape: pl.BlockSpec(shape, lambda i: (0, 0))
    slab = lambda: pl.BlockSpec((MB, 128), lambda i: (i, 0))
    return pl.pallas_call(
        _conv2_body,
        grid=(M // MB,),
        in_specs=(
            [pl.BlockSpec((2, MB, 128), lambda i: (0, i, 0))
             for _ in range(4)]
            + [slab() for _ in range(4)]
            + [full((500, 256)), full((500, 256)), full((1, 256))]
        ),
        out_specs=[slab(), slab()],
        out_shape=[jax.ShapeDtypeStruct((M, 128), jnp.float32)
                   for _ in range(2)],
    )(*aggps, *hs, jnp.pad(W_rel2, ((0, 0), (0, 56))),
      jnp.pad(W_root2, ((0, 0), (0, 56))),
      jnp.pad(b_rel2, (0, 56)).reshape(1, -1))


def _final_body(poolpa, poolpb, cntp, c, wmol, bmol, wc1, bc1, wc2, bc2, wc3,
                bc3, out):
    sums = jnp.concatenate(
        [poolpa[0] + poolpa[1], poolpb[0, :, 0:72] + poolpb[1, :, 0:72]],
        axis=1)
    cnt = cntp[0, :, 0:1] + cntp[1, :, 0:1]
    pooled = sums / jnp.maximum(cnt, 1.0)
    m = _dot(pooled, wmol[...]) + bmol[...]
    xcat = jnp.concatenate([m, c[...]], axis=1)
    z = jnp.maximum(_dot(xcat, wc1[...]) + bc1[...], 0.0)
    z = jnp.maximum(_dot(z, wc2[...]) + bc2[...], 0.0)
    out[...] = _dot(z, wc3[...]) + bc3[...]


def _final(poolpa, poolpb, cntp, c, W_mol, b_mol, Wc1, bc1, Wc2, bc2, Wc3, bc3):
    return pl.pallas_call(
        _final_body,
        out_shape=jax.ShapeDtypeStruct((_N_GRAPHS, 1), jnp.float32),
    )(poolpa, poolpb, cntp, c, W_mol, b_mol.reshape(1, -1),
      Wc1, bc1.reshape(1, -1), Wc2, bc2.reshape(1, -1),
      Wc3, bc3.reshape(1, -1))


def kernel(cll, x_mol, edge_index, batch,
           W_rel1, b_rel1, W_root1,
           W_rel2, b_rel2, W_root2,
           W_mol, b_mol,
           W1, b1, W2, b2, W3, b3, W4, b4,
           Wc1, bc1, Wc2, bc2, Wc3, bc3):
    src = edge_index[0]
    dst = edge_index[1]

    c = _cll_mlp(cll, W1, b1, W2, b2, W3, b3, W4, b4)

    z128 = jnp.zeros((640, 128), jnp.float32)
    ones128 = jnp.ones((_ECH, 128), jnp.float32)

    (agg1p,) = _make_edge_agg(1)(x_mol, src, dst, z128)
    Wcat1 = jnp.concatenate([W_rel1, W_root1], axis=0)
    hs = _conv1(agg1p, x_mol, Wcat1, b_rel1)

    aggps = _make_edge_agg(4)(*hs, src, dst, z128)
    h2a, h2b = _conv2(aggps, hs, W_rel2, W_root2, b_rel2)

    poolpa, poolpb, cntp = _make_pool()(h2a, h2b, batch, z128[:64], ones128)

    return _final(poolpa, poolpb, cntp, c, W_mol, b_mol, Wc1, bc1, Wc2, bc2,
                  Wc3, bc3)
